# Initial kernel scaffold; baseline (speedup 1.0000x reference)
import jax
import jax.numpy as jnp
from jax.experimental import pallas as pl


def kernel(pred0, pred1, pred2, boxes, labels):
    raise NotImplementedError("write your pallas kernel here")


# speedup vs baseline: 1.7425x; 1.7425x over previous
import functools
import jax
import jax.numpy as jnp
from jax import lax
from jax.experimental import pallas as pl
from jax.experimental.pallas import tpu as pltpu
from jax.experimental.pallas import tpu_sc as plsc

_B = 16
_T = 32
_NT = _B * _T
_NC2 = 2
_NS = 16
_NW = _NC2 * _NS
_TPW = _NT // _NW
_NCH = 85
_SCALES = ((64, 8.0), (32, 16.0), (16, 32.0))
_ANCHORS = (((10.0, 13.0), (16.0, 30.0), (33.0, 23.0)),
            ((30.0, 61.0), (62.0, 45.0), (59.0, 119.0)),
            ((116.0, 90.0), (156.0, 198.0), (373.0, 326.0)))
_EPS = 1e-7


def _log1p01(z):
    s = z / (z + 2.0)
    s2 = s * s
    return 2.0 * s * (1.0 + s2 * (1.0/3.0 + s2 * (1.0/5.0
                      + s2 * (1.0/7.0 + s2 * (1.0/9.0)))))


def _sigmoid(x):
    return 1.0 / (1.0 + jnp.exp(-x))


def _scales_acc(p0, p1, p2, bxv, byv, bwv, bhv, labv, idxv, valv, sem, base):
    bx = bxv[...]
    by = byv[...]
    bw = bwv[...]
    bh = bhv[...]
    lab = labv[...]
    g = base + lax.iota(jnp.int32, _TPW)
    bidx = lax.shift_right_logical(g, 5)

    acc = jnp.zeros((_TPW,), jnp.float32)
    preds = (p0, p1, p2)
    for si, (n, st) in enumerate(_SCALES):
        n2 = n * n
        nf = float(n)
        tgx = bx * nf
        tgy = by * nf
        tgw = bw * nf
        tgh = bh * nf
        gi = jnp.minimum(jnp.maximum(tgx.astype(jnp.int32), 0), n - 1)
        gj = jnp.minimum(jnp.maximum(tgy.astype(jnp.int32), 0), n - 1)
        gif = gi.astype(jnp.float32)
        gjf = gj.astype(jnp.float32)

        a = jnp.zeros((_TPW,), jnp.int32)
        rbest = None
        awv = None
        ahv = None
        for k in range(3):
            aw = _ANCHORS[si][k][0] / st
            ah = _ANCHORS[si][k][1] / st
            rw = jnp.maximum(tgw / aw, aw / tgw)
            rh = jnp.maximum(tgh / ah, ah / tgh)
            r = jnp.maximum(rw, rh)
            if k == 0:
                rbest = r
                awv = jnp.full((_TPW,), aw, jnp.float32)
                ahv = jnp.full((_TPW,), ah, jnp.float32)
            else:
                bet = r < rbest
                rbest = jnp.where(bet, r, rbest)
                a = jnp.where(bet, k, a)
                awv = jnp.where(bet, aw, awv)
                ahv = jnp.where(bet, ah, ahv)

        flat0 = ((bidx * 255 + a * _NCH) * n + gj) * n + gi

        def build(ci, carry):
            idxv[pl.ds(ci * _TPW, _TPW)] = flat0 + ci * n2
            return carry
        lax.fori_loop(0, _NCH, build, 0)
        pltpu.async_copy(preds[si].at[idxv], valv, sem).wait()

        v0 = valv[pl.ds(0 * _TPW, _TPW)]
        v1 = valv[pl.ds(1 * _TPW, _TPW)]
        v2 = valv[pl.ds(2 * _TPW, _TPW)]
        v3 = valv[pl.ds(3 * _TPW, _TPW)]
        v4 = valv[pl.ds(4 * _TPW, _TPW)]
        sx = _sigmoid(v0)
        sy = _sigmoid(v1)
        ew = jnp.exp(v2) * awv
        eh = jnp.exp(v3) * ahv
        px = sx + gif
        py = sy + gjf
        tx = tgx - gif
        ty = tgy - gjf
        b1x1 = px - ew * 0.5
        b1x2 = px + ew * 0.5
        b1y1 = py - eh * 0.5
        b1y2 = py + eh * 0.5
        b2x1 = tx - tgw * 0.5
        b2x2 = tx + tgw * 0.5
        b2y1 = ty - tgh * 0.5
        b2y2 = ty + tgh * 0.5
        inter = (jnp.maximum(jnp.minimum(b1x2, b2x2) - jnp.maximum(b1x1, b2x1), 0.0)
                 * jnp.maximum(jnp.minimum(b1y2, b2y2) - jnp.maximum(b1y1, b2y1), 0.0))
        union = ew * (eh + _EPS) + tgw * (tgh + _EPS) - inter + _EPS
        iou = inter / union
        cw = jnp.maximum(b1x2, b2x2) - jnp.minimum(b1x1, b2x1)
        chh = jnp.maximum(b1y2, b2y2) - jnp.minimum(b1y1, b2y1)
        c2 = cw * cw + chh * chh + _EPS
        dx = b2x1 + b2x2 - b1x1 - b1x2
        dy = b2y1 + b2y2 - b1y1 - b1y2
        rho2 = (dx * dx + dy * dy) * 0.25
        diou = iou - rho2 / c2
        acc = acc + 5.0 * (1.0 - diou)

        so = _sigmoid(v4)
        acc = acc + _log1p01(jnp.exp(-so))

        def cls_body(ci, acc_c):
            sc = _sigmoid(valv[pl.ds((5 + ci) * _TPW, _TPW)])
            y = jnp.where(lab == ci, 1.0, 0.0)
            return acc_c + sc * (1.0 - y) + _log1p01(jnp.exp(-sc))
        acc_cls = lax.fori_loop(0, 80, cls_body, jnp.zeros((_TPW,), jnp.float32))
        acc = acc + acc_cls * (1.0 / 80.0)
    return acc


def _sc_body(p0, p1, p2, bx_h, by_h, bw_h, bh_h, lab_h, out_h,
             bxv, byv, bwv, bhv, labv, idxv, valv, accv, sem):
    cid = lax.axis_index("c")
    sid = lax.axis_index("s")
    wid = cid * _NS + sid
    base = wid * _TPW

    pltpu.sync_copy(bx_h.at[pl.ds(base, _TPW)], bxv)
    pltpu.sync_copy(by_h.at[pl.ds(base, _TPW)], byv)
    pltpu.sync_copy(bw_h.at[pl.ds(base, _TPW)], bwv)
    pltpu.sync_copy(bh_h.at[pl.ds(base, _TPW)], bhv)
    pltpu.sync_copy(lab_h.at[pl.ds(base, _TPW)], labv)

    acc = _scales_acc(p0, p1, p2, bxv, byv, bwv, bhv, labv, idxv, valv,
                      sem, base)
    accv[...] = acc
    pltpu.sync_copy(accv, out_h.at[wid])


_sc_scratch = [
    pltpu.VMEM((_TPW,), jnp.float32),
    pltpu.VMEM((_TPW,), jnp.float32),
    pltpu.VMEM((_TPW,), jnp.float32),
    pltpu.VMEM((_TPW,), jnp.float32),
    pltpu.VMEM((_TPW,), jnp.int32),
    pltpu.VMEM((_NCH * _TPW,), jnp.int32),
    pltpu.VMEM((_NCH * _TPW,), jnp.float32),
    pltpu.VMEM((_TPW,), jnp.float32),
    pltpu.SemaphoreType.DMA,
]

_yolo_sc = pl.kernel(
    _sc_body,
    out_type=jax.ShapeDtypeStruct((_NW, _TPW), jnp.float32),
    mesh=plsc.VectorSubcoreMesh(core_axis_name="c", subcore_axis_name="s"),
    scratch_types=_sc_scratch,
)


@jax.jit
def kernel(pred0, pred1, pred2, boxes, labels):
    p0 = pred0.reshape(-1)
    p1 = pred1.reshape(-1)
    p2 = pred2.reshape(-1)
    bf = boxes.reshape(_NT, 4)
    lab = labels.reshape(_NT).astype(jnp.int32)
    out = _yolo_sc(p0, p1, p2, bf[:, 0], bf[:, 1], bf[:, 2], bf[:, 3], lab)
    return jnp.sum(out).reshape(1)
